# baseline (speedup 1.0000x reference)
import jax
import jax.numpy as jnp
from jax import lax
from jax.experimental import pallas as pl
from jax.experimental.pallas import tpu as pltpu
from jax.experimental.pallas import tpu_sc as plsc

R = 128
C = 32768
NC = 2
NS = 16
NW = NC * NS
RPW = R // NW
L = 16
NV = C // L
NB = 16
KV = NV // NB
ACC = 4


def _shuffle(v, idx):
    dnums = lax.GatherDimensionNumbers(
        offset_dims=(), collapsed_slice_dims=(0,), start_index_map=(0,))
    return lax.gather(v, idx[:, None], dnums, (1,),
                      mode=lax.GatherScatterMode.PROMISE_IN_BOUNDS)


def _bfly(v, op):
    lane = lax.iota(jnp.int32, L)
    for s in (8, 4, 2, 1):
        v = op(v, _shuffle(v, lane ^ s))
    return v


def _scan_row(buf):
    lane = lax.iota(jnp.int32, L)
    neg = jnp.full((L,), -jnp.inf, jnp.float32)
    big = jnp.full((L,), C, jnp.int32)

    bms = []
    for b in range(NB):
        base = b * KV * L

        def it(i, accs):
            off = base + i * (ACC * L)
            vs = [buf[pl.ds(off + k * L, L)] for k in range(ACC)]
            return tuple(jnp.maximum(a, v) for a, v in zip(accs, vs))

        accs = lax.fori_loop(0, KV // ACC, it, (neg,) * ACC, unroll=4)
        bms.append(jnp.maximum(jnp.maximum(accs[0], accs[1]),
                               jnp.maximum(accs[2], accs[3])))

    g = bms[0]
    for b in range(1, NB):
        g = jnp.maximum(g, bms[b])
    m = _bfly(g, jnp.maximum)

    fb = jnp.full((L,), NB, jnp.int32)
    for b in range(NB):
        fb = jnp.minimum(fb, jnp.where(bms[b] == m, b, NB))
    bstar = _bfly(fb, jnp.minimum)[0]

    ebase = bstar * (KV * L)

    def it3(i, carry):
        fi, iv = carry
        v = buf[pl.ds(ebase + i * L, L)]
        fi = jnp.minimum(fi, jnp.where(v == m, iv, big))
        return fi, iv + L

    fi, _ = lax.fori_loop(0, KV, it3, (big, lane + ebase), unroll=4)
    return _bfly(fi, jnp.minimum)


def _body(x_hbm, out_hbm, buf0, buf1, res, sem0, sem1):
    cid = lax.axis_index("c")
    sid = lax.axis_index("s")
    wid = sid * NC + cid
    r0 = wid * RPW

    bufs = (buf0, buf1)
    sems = (sem0, sem1)

    cps = [pltpu.async_copy(x_hbm.at[r0 + j], bufs[j], sems[j])
           for j in range(2)]
    for j in range(RPW):
        b = j % 2
        cps[b].wait()
        amax = _scan_row(bufs[b])
        if j + 2 < RPW:
            cps[b] = pltpu.async_copy(x_hbm.at[r0 + j + 2], bufs[b], sems[b])
        res[j, :] = amax

    pltpu.sync_copy(res, out_hbm.at[wid])


@jax.jit
def _argmax_sc(x):
    mesh = plsc.VectorSubcoreMesh(core_axis_name="c", subcore_axis_name="s")
    k = pl.kernel(
        _body,
        mesh=mesh,
        out_type=jax.ShapeDtypeStruct((NW, RPW, L), jnp.int32),
        scratch_types=[
            pltpu.VMEM((C,), jnp.float32),
            pltpu.VMEM((C,), jnp.float32),
            pltpu.VMEM((RPW, L), jnp.int32),
            pltpu.SemaphoreType.DMA,
            pltpu.SemaphoreType.DMA,
        ],
    )
    out = k(x)
    return out.reshape(R, L)[:, 0]


def kernel(x):
    return _argmax_sc(x)


# speedup vs baseline: 1.1463x; 1.1463x over previous
import jax
import jax.numpy as jnp
from jax import lax
from jax.experimental import pallas as pl
from jax.experimental.pallas import tpu as pltpu
from jax.experimental.pallas import tpu_sc as plsc

R = 128
C = 32768
NC = 2
NS = 16
NW = NC * NS
RPW = R // NW
L = 16
NV = C // L
NB = 16
KV = NV // NB
ACC = 4


def _shuffle(v, idx):
    dnums = lax.GatherDimensionNumbers(
        offset_dims=(), collapsed_slice_dims=(0,), start_index_map=(0,))
    return lax.gather(v, idx[:, None], dnums, (1,),
                      mode=lax.GatherScatterMode.PROMISE_IN_BOUNDS)


def _bfly(v, op):
    lane = lax.iota(jnp.int32, L)
    for s in (8, 4, 2, 1):
        v = op(v, _shuffle(v, lane ^ s))
    return v


def _scan_row(buf, bm_ref):
    lane = lax.iota(jnp.int32, L)
    neg = jnp.full((L,), -jnp.inf, jnp.float32)
    big = jnp.full((L,), C, jnp.int32)

    def blk(b, g):
        base = b * (KV * L)

        def it(i, accs):
            off = base + i * (ACC * L)
            vs = [buf[pl.ds(off + k * L, L)] for k in range(ACC)]
            return tuple(jnp.maximum(a, v) for a, v in zip(accs, vs))

        accs = lax.fori_loop(0, KV // ACC, it, (neg,) * ACC, unroll=2)
        bm = jnp.maximum(jnp.maximum(accs[0], accs[1]),
                         jnp.maximum(accs[2], accs[3]))
        bm_ref[pl.ds(b * L, L)] = bm
        return jnp.maximum(g, bm)

    g = lax.fori_loop(0, NB, blk, neg)
    m = _bfly(g, jnp.maximum)

    def fbit(b, fb):
        bmb = bm_ref[pl.ds(b * L, L)]
        return jnp.minimum(fb, jnp.where(bmb == m, b, NB))

    fb = lax.fori_loop(0, NB, fbit, jnp.full((L,), NB, jnp.int32))
    bstar = _bfly(fb, jnp.minimum)[0]

    ebase = bstar * (KV * L)

    def it3(i, carry):
        fi, iv = carry
        v = buf[pl.ds(ebase + i * L, L)]
        fi = jnp.minimum(fi, jnp.where(v == m, iv, big))
        return fi, iv + L

    fi, _ = lax.fori_loop(0, KV, it3, (big, lane + ebase), unroll=2)
    return _bfly(fi, jnp.minimum)


def _body(x_hbm, out_hbm, buf0, buf1, res, bm_ref, sem0, sem1):
    cid = lax.axis_index("c")
    sid = lax.axis_index("s")
    wid = sid * NC + cid
    r0 = wid * RPW

    bufs = (buf0, buf1)
    sems = (sem0, sem1)

    cps = [pltpu.async_copy(x_hbm.at[r0 + j], bufs[j], sems[j])
           for j in range(2)]
    for j in range(RPW):
        b = j % 2
        cps[b].wait()
        amax = _scan_row(bufs[b], bm_ref)
        if j + 2 < RPW:
            cps[b] = pltpu.async_copy(x_hbm.at[r0 + j + 2], bufs[b], sems[b])
        res[j, :] = amax

    pltpu.sync_copy(res, out_hbm.at[wid])


@jax.jit
def _argmax_sc(x):
    mesh = plsc.VectorSubcoreMesh(core_axis_name="c", subcore_axis_name="s")
    k = pl.kernel(
        _body,
        mesh=mesh,
        out_type=jax.ShapeDtypeStruct((NW, RPW, L), jnp.int32),
        scratch_types=[
            pltpu.VMEM((C,), jnp.float32),
            pltpu.VMEM((C,), jnp.float32),
            pltpu.VMEM((RPW, L), jnp.int32),
            pltpu.VMEM((NB * L,), jnp.float32),
            pltpu.SemaphoreType.DMA,
            pltpu.SemaphoreType.DMA,
        ],
    )
    out = k(x)
    return out.reshape(R, L)[:, 0]


def kernel(x):
    return _argmax_sc(x)
